# baseline (speedup 1.0000x reference)
import functools

import jax
import jax.numpy as jnp
from jax import lax
from jax.experimental import pallas as pl
from jax.experimental.pallas import tpu as pltpu
from jax.experimental.pallas import tpu_sc as plsc

_INFO = plsc.get_sparse_core_info()
_NC = _INFO.num_cores
_NS = _INFO.num_subcores
_NW = _NC * _NS

_CHUNK = 128
_G = 4


@functools.partial(jax.jit, static_argnums=(2, 3))
def _sc_gather(table, idx3, n_rows, embed_dim):
    n_chunks = idx3.shape[0] // _NW
    steps = n_chunks // _G
    rows_per_step = _G * _CHUNK
    b_per_w = n_chunks * _CHUNK

    mesh = plsc.VectorSubcoreMesh(core_axis_name="c", subcore_axis_name="s")

    @functools.partial(
        pl.kernel,
        mesh=mesh,
        out_type=jax.ShapeDtypeStruct((n_rows, embed_dim), jnp.float32),
        scratch_types=[
            pltpu.VMEM((n_chunks, _CHUNK), jnp.int32),
            pltpu.VMEM((rows_per_step, embed_dim), jnp.float32),
            pltpu.SemaphoreType.DMA,
        ],
        compiler_params=pltpu.CompilerParams(use_tc_tiling_on_sc=False),
    )
    def k(table_hbm, idx_hbm, out_hbm, idx_v, rows_v, gsem):
        wid = lax.axis_index("s") * _NC + lax.axis_index("c")
        cbase = wid * n_chunks
        rbase = wid * b_per_w
        pltpu.sync_copy(idx_hbm.at[pl.ds(cbase, n_chunks)], idx_v)

        def step(s, carry):
            copies = []
            for b in range(_G):
                g = s * _G + b
                copies.append(
                    pltpu.async_copy(
                        table_hbm.at[idx_v.at[g]],
                        rows_v.at[pl.ds(b * _CHUNK, _CHUNK)],
                        gsem,
                    )
                )
            for c in copies:
                c.wait()
            pltpu.sync_copy(
                rows_v, out_hbm.at[pl.ds(rbase + s * rows_per_step, rows_per_step)]
            )
            return carry

        lax.fori_loop(0, steps, step, 0)

    return k(table, idx3)


def kernel(x, table):
    batch, n_fields = x.shape
    embed_dim = table.shape[1]
    n_rows = batch * n_fields
    idx3 = x.reshape(-1).astype(jnp.int32).reshape(n_rows // _CHUNK, _CHUNK)
    out = _sc_gather(table, idx3, n_rows, embed_dim)
    return out.reshape(batch, n_fields, embed_dim)


# speedup vs baseline: 1.0132x; 1.0132x over previous
import functools

import jax
import jax.numpy as jnp
from jax import lax
from jax.experimental import pallas as pl
from jax.experimental.pallas import tpu as pltpu
from jax.experimental.pallas import tpu_sc as plsc

_INFO = plsc.get_sparse_core_info()
_NC = _INFO.num_cores
_NS = _INFO.num_subcores
_NW = _NC * _NS

_CHUNK = 128
_G = 4


@functools.partial(jax.jit, static_argnums=(2, 3))
def _sc_gather(table, idx3, n_rows, embed_dim):
    n_chunks = idx3.shape[0] // _NW
    steps = n_chunks // _G
    rows_per_step = _G * _CHUNK
    b_per_w = n_chunks * _CHUNK

    mesh = plsc.VectorSubcoreMesh(core_axis_name="c", subcore_axis_name="s")

    @functools.partial(
        pl.kernel,
        mesh=mesh,
        out_type=jax.ShapeDtypeStruct((n_rows, embed_dim), jnp.float32),
        scratch_types=[
            pltpu.VMEM((n_chunks, _CHUNK), jnp.int32),
            pltpu.VMEM((2, rows_per_step, embed_dim), jnp.float32),
            pltpu.SemaphoreType.DMA,
            pltpu.SemaphoreType.DMA,
            pltpu.SemaphoreType.DMA,
        ],
        compiler_params=pltpu.CompilerParams(use_tc_tiling_on_sc=False),
    )
    def k(table_hbm, idx_hbm, out_hbm, idx_v, rows_v, gsem, osem0, osem1):
        wid = lax.axis_index("s") * _NC + lax.axis_index("c")
        cbase = wid * n_chunks
        rbase = wid * b_per_w
        osems = (osem0, osem1)
        pltpu.sync_copy(idx_hbm.at[pl.ds(cbase, n_chunks)], idx_v)

        def gather_into(s, buf):
            copies = []
            for b in range(_G):
                copies.append(
                    pltpu.async_copy(
                        table_hbm.at[idx_v.at[s * _G + b]],
                        rows_v.at[buf, pl.ds(b * _CHUNK, _CHUNK)],
                        gsem,
                    )
                )
            return copies

        def write_out(s, buf):
            return pltpu.async_copy(
                rows_v.at[buf],
                out_hbm.at[pl.ds(rbase + s * rows_per_step, rows_per_step)],
                osems[buf],
            )

        def wait_write(buf):
            pltpu.make_async_copy(
                rows_v.at[buf],
                out_hbm.at[pl.ds(rbase, rows_per_step)],
                osems[buf],
            ).wait()

        for buf in range(2):
            for c in gather_into(buf, buf):
                c.wait()
            write_out(buf, buf)

        def body(s2, carry):
            for buf in range(2):
                s = 2 * s2 + buf
                wait_write(buf)
                for c in gather_into(s, buf):
                    c.wait()
                write_out(s, buf)
            return carry

        lax.fori_loop(1, steps // 2, body, 0)

        wait_write(0)
        wait_write(1)

    return k(table, idx3)


def kernel(x, table):
    batch, n_fields = x.shape
    embed_dim = table.shape[1]
    n_rows = batch * n_fields
    idx3 = x.reshape(-1).astype(jnp.int32).reshape(n_rows // _CHUNK, _CHUNK)
    out = _sc_gather(table, idx3, n_rows, embed_dim)
    return out.reshape(batch, n_fields, embed_dim)
